# baseline (speedup 1.0000x reference)
import jax
import jax.numpy as jnp
from jax import lax
from jax.experimental import pallas as pl
from jax.experimental.pallas import tpu as pltpu
from jax.experimental.pallas import tpu_sc as plsc

NUM_CORES = 2
NUM_SUBCORES = 16
NUM_WORKERS = NUM_CORES * NUM_SUBCORES

BATCH = 16384
PER_TEAM = 50
EMBED_DIM = 64
LANES = 16
B_PER_WORKER = BATCH // NUM_WORKERS
CB = 256
HALVES = B_PER_WORKER // CB
NBUF = 2
NUNITS = PER_TEAM * HALVES
NGROUPS = NUNITS // NBUF


def _gather_kernel(table_hbm, idxt_hbm, out_hbm, idx_v, dense_v, rowst_v,
                   gsems, wsems):
    wid = lax.axis_index("s") * NUM_CORES + lax.axis_index("c")
    b0 = wid * B_PER_WORKER

    pltpu.sync_copy(idxt_hbm.at[:, pl.ds(b0, B_PER_WORKER)], idx_v)

    def fire_gather(u, b):
        p, h = u // HALVES, u % HALVES
        pltpu.async_copy(table_hbm.at[idx_v.at[p, pl.ds(h * CB, CB)]],
                         dense_v.at[b], gsems.at[b])

    def wait_gather(b):
        pltpu.make_async_copy(table_hbm.at[pl.ds(0, CB)], dense_v.at[b],
                              gsems.at[b]).wait()

    iota = lax.broadcasted_iota(jnp.int32, (LANES,), 0)
    rows_js = [iota + js * LANES for js in range(CB // LANES)]

    def transpose(b):
        def body(i, dvec):
            for js in range(CB // LANES):
                v = plsc.load_gather(dense_v.at[b], [rows_js[js], dvec])
                rowst_v[b, i, pl.ds(js * LANES, LANES)] = v
            return dvec + 1

        lax.fori_loop(0, EMBED_DIM, body, jnp.zeros((LANES,), jnp.int32),
                      unroll=False)

    def fire_wb(u, b):
        p, h = u // HALVES, u % HALVES
        pltpu.async_copy(rowst_v.at[b],
                         out_hbm.at[p, :, pl.ds(b0 + h * CB, CB)], wsems.at[b])

    def wait_wb(b):
        pltpu.make_async_copy(rowst_v.at[b], out_hbm.at[0, :, pl.ds(0, CB)],
                              wsems.at[b]).wait()

    for b in range(NBUF):
        fire_gather(b, b)

    def group(g, carry):
        for b in range(NBUF):
            u = g * NBUF + b
            wait_gather(b)
            wait_wb(b)
            transpose(b)
            fire_wb(u, b)
            fire_gather(u + NBUF, b)
        return carry

    for b in range(NBUF):
        wait_gather(b)
        transpose(b)
        fire_wb(b, b)
        fire_gather(b + NBUF, b)

    def group1(g, carry):
        return group(g + 1, carry)

    lax.fori_loop(0, NGROUPS - 2, group1, 0, unroll=False)

    for b in range(NBUF):
        u = (NGROUPS - 1) * NBUF + b
        wait_gather(b)
        wait_wb(b)
        transpose(b)
        fire_wb(u, b)
    for b in range(NBUF):
        wait_wb(b)


@jax.jit
def _embed(champion_ids, table):
    mesh = plsc.VectorSubcoreMesh(core_axis_name="c", subcore_axis_name="s")
    run = pl.kernel(
        _gather_kernel,
        out_type=jax.ShapeDtypeStruct((PER_TEAM, EMBED_DIM, BATCH),
                                      jnp.float32),
        mesh=mesh,
        scratch_types=[
            pltpu.VMEM((PER_TEAM, B_PER_WORKER), jnp.int32),
            pltpu.VMEM((NBUF, CB, EMBED_DIM), jnp.float32),
            pltpu.VMEM((NBUF, EMBED_DIM, CB), jnp.float32),
            pltpu.SemaphoreType.DMA((NBUF,)),
            pltpu.SemaphoreType.DMA((NBUF,)),
        ],
        compiler_params=pltpu.CompilerParams(use_tc_tiling_on_sc=False,
                                             needs_layout_passes=False),
    )
    ids_t = champion_ids.astype(jnp.int32).T
    x = run(table, ids_t)
    return jnp.transpose(x, (2, 0, 1))


def kernel(champion_ids, table):
    return _embed(champion_ids, table)


# speedup vs baseline: 2.2545x; 2.2545x over previous
import jax
import jax.numpy as jnp
from jax import lax
from jax.experimental import pallas as pl
from jax.experimental.pallas import tpu as pltpu
from jax.experimental.pallas import tpu_sc as plsc

NUM_CORES = 2
NUM_SUBCORES = 16
NUM_WORKERS = NUM_CORES * NUM_SUBCORES

BATCH = 16384
PER_TEAM = 50
EMBED_DIM = 64
B_PER_WORKER = BATCH // NUM_WORKERS
CB = 256
HALVES = B_PER_WORKER // CB
NBUF = 4
NUNITS = PER_TEAM * HALVES
NGROUPS = NUNITS // NBUF


def _gather_kernel(table_hbm, idxt_hbm, out_hbm, idx_v, dense_v, gsems,
                   wsems):
    wid = lax.axis_index("s") * NUM_CORES + lax.axis_index("c")
    b0 = wid * B_PER_WORKER

    pltpu.sync_copy(idxt_hbm.at[:, pl.ds(b0, B_PER_WORKER)], idx_v)

    def fire_gather(u, b):
        p, h = u // HALVES, u % HALVES
        pltpu.async_copy(table_hbm.at[idx_v.at[p, pl.ds(h * CB, CB)]],
                         dense_v.at[b], gsems.at[b])

    def wait_gather(b):
        pltpu.make_async_copy(table_hbm.at[pl.ds(0, CB)], dense_v.at[b],
                              gsems.at[b]).wait()

    def fire_wb(u, b):
        p, h = u // HALVES, u % HALVES
        pltpu.async_copy(dense_v.at[b],
                         out_hbm.at[p, pl.ds(b0 + h * CB, CB)], wsems.at[b])

    def wait_wb(b):
        pltpu.make_async_copy(dense_v.at[b], out_hbm.at[0, pl.ds(0, CB)],
                              wsems.at[b]).wait()

    for b in range(NBUF):
        fire_gather(b, b)

    def group(g, carry):
        for b in range(NBUF):
            u = g * NBUF + b
            wait_gather(b)
            fire_wb(u, b)
            wait_wb(b)
            fire_gather(u + NBUF, b)
        return carry

    lax.fori_loop(0, NGROUPS - 1, group, 0, unroll=False)

    for b in range(NBUF):
        u = (NGROUPS - 1) * NBUF + b
        wait_gather(b)
        fire_wb(u, b)
    for b in range(NBUF):
        wait_wb(b)


@jax.jit
def _embed(champion_ids, table):
    mesh = plsc.VectorSubcoreMesh(core_axis_name="c", subcore_axis_name="s")
    run = pl.kernel(
        _gather_kernel,
        out_type=jax.ShapeDtypeStruct((PER_TEAM, BATCH, EMBED_DIM),
                                      jnp.float32),
        mesh=mesh,
        scratch_types=[
            pltpu.VMEM((PER_TEAM, B_PER_WORKER), jnp.int32),
            pltpu.VMEM((NBUF, CB, EMBED_DIM), jnp.float32),
            pltpu.SemaphoreType.DMA((NBUF,)),
            pltpu.SemaphoreType.DMA((NBUF,)),
        ],
        compiler_params=pltpu.CompilerParams(use_tc_tiling_on_sc=False),
    )
    ids_t = champion_ids.astype(jnp.int32).T
    x = run(table, ids_t)
    return jnp.transpose(x, (1, 0, 2))


def kernel(champion_ids, table):
    return _embed(champion_ids, table)
